# baseline (speedup 1.0000x reference)
import functools

import jax
import jax.numpy as jnp
from jax import lax
from jax.experimental import pallas as pl
from jax.experimental.pallas import tpu as pltpu
from jax.experimental.pallas import tpu_sc as plsc

DIM = 32
BATCH = 16384
L = 16

_info = plsc.get_sparse_core_info()
NC, NS = _info.num_cores, _info.num_subcores
NW = NC * NS
BPW = BATCH // NW


def _rsqrt(x):
    i = lax.bitcast_convert_type(x, jnp.int32)
    i = jnp.int32(0x5F3759DF) - lax.shift_right_arithmetic(i, 1)
    y = lax.bitcast_convert_type(i, jnp.float32)
    for _ in range(3):
        y = y * (1.5 - 0.5 * x * y * y)
    return y


@functools.partial(
    pl.kernel,
    mesh=plsc.VectorSubcoreMesh(core_axis_name="c", subcore_axis_name="s"),
    out_type=jax.ShapeDtypeStruct((BATCH, DIM), jnp.float32),
    compiler_params=pltpu.CompilerParams(use_tc_tiling_on_sc=False),
    scratch_types=[
        pltpu.VMEM((BPW,), jnp.int32),
        pltpu.VMEM((BPW,), jnp.int32),
        pltpu.VMEM((BPW, DIM), jnp.float32),
        pltpu.VMEM((BPW, DIM), jnp.float32),
        pltpu.VMEM((DIM,), jnp.float32),
        pltpu.VMEM((DIM,), jnp.float32),
        pltpu.SemaphoreType.DMA,
        pltpu.SemaphoreType.DMA,
    ],
)
def _sc_embed_ln(i0_hbm, i1_hbm, t0_hbm, t1_hbm, gamma_hbm, beta_hbm,
                 out_hbm, idx0_v, idx1_v, rows0_v, rows1_v, g_v, b_v,
                 sem0, sem1):
    wid = lax.axis_index("s") * NC + lax.axis_index("c")
    base = wid * BPW

    pltpu.sync_copy(i0_hbm.at[pl.ds(base, BPW)], idx0_v)
    pltpu.sync_copy(i1_hbm.at[pl.ds(base, BPW)], idx1_v)
    c0 = pltpu.async_copy(t0_hbm.at[idx0_v], rows0_v, sem0)
    c1 = pltpu.async_copy(t1_hbm.at[idx1_v], rows1_v, sem1)
    pltpu.sync_copy(gamma_hbm, g_v)
    pltpu.sync_copy(beta_hbm, b_v)
    c0.wait()
    c1.wait()

    g0 = g_v[pl.ds(0, L)]
    g1 = g_v[pl.ds(L, L)]
    bb0 = b_v[pl.ds(0, L)]
    bb1 = b_v[pl.ds(L, L)]

    lanes = lax.iota(jnp.int32, L)
    perms = [(lanes ^ (1 << k))[:, None] for k in range(4)]
    dn = lax.GatherDimensionNumbers(
        offset_dims=(), collapsed_slice_dims=(0,), start_index_map=(0,))

    def allsum(v):
        for p in perms:
            v = v + lax.gather(v, p, dn, slice_sizes=(1,),
                               mode=lax.GatherScatterMode.PROMISE_IN_BOUNDS)
        return v

    def body(i, carry):
        ea = rows0_v[i, pl.ds(0, L)] + rows1_v[i, pl.ds(0, L)]
        eb = rows0_v[i, pl.ds(L, L)] + rows1_v[i, pl.ds(L, L)]
        mean = allsum(ea + eb) * (1.0 / DIM)
        da = ea - mean
        db = eb - mean
        var = allsum(da * da + db * db) * (1.0 / DIM) + 1e-5
        r = _rsqrt(var)
        rows0_v[i, pl.ds(0, L)] = da * r * g0 + bb0
        rows0_v[i, pl.ds(L, L)] = db * r * g1 + bb1
        return carry

    lax.fori_loop(0, BPW, body, 0)
    pltpu.sync_copy(rows0_v, out_hbm.at[pl.ds(base, BPW), :])


def kernel(inputs, T0, T1, gamma, beta):
    return _sc_embed_ln(inputs[0], inputs[1], T0, T1, gamma, beta)
